# baseline (speedup 1.0000x reference)
import jax
import jax.numpy as jnp
from jax import lax
from jax.experimental import pallas as pl
from jax.experimental.pallas import tpu as pltpu
from jax.experimental.pallas import tpu_sc as plsc

NUM_CORES = 2
NUM_SUBCORES = 16
NW = NUM_CORES * NUM_SUBCORES
L = 16
BATCH = 16384
EMB = 64
BPW = BATCH // NW
NIDX = 4
IDXW = BPW // NIDX
CHUNKS = BPW // L
KREG = EMB // L


def _sig(x):
    return 1.0 / (1.0 + jnp.exp(-x))


def _mf_body(fld_hbm, user_emb, user_bias, item_emb, item_bias,
             mean_hbm, svar_hbm, out_hbm,
             fld_v, idx_u, idx_v, tiles_u, tiles_v, subs_u, subs_v,
             u_rows, i_rows, bu, bi, mean_v, sv_v, out_v, sem):
    wid = lax.axis_index("s") * NUM_CORES + lax.axis_index("c")

    pltpu.sync_copy(fld_hbm.at[pl.ds(wid * (2 * BPW), 2 * BPW)], fld_v)
    pltpu.sync_copy(mean_hbm, mean_v)
    pltpu.sync_copy(svar_hbm, sv_v)

    lanes = lax.iota(jnp.int32, L)
    zeros16 = jnp.zeros((L,), jnp.int32)
    mask15 = lanes == (L - 1)
    cols = [lanes + (k * L) for k in range(KREG)]

    for g in range(BPW // L):
        even = 2 * (lanes + g * L)
        uid = plsc.load_gather(fld_v, [even])
        iid = plsc.load_gather(fld_v, [even + 1])
        idx_u[g // 8, pl.ds((g % 8) * L, L)] = uid
        idx_v[g // 8, pl.ds((g % 8) * L, L)] = iid
        tiles_u[g // 8, pl.ds((g % 8) * L, L)] = uid >> 1
        tiles_v[g // 8, pl.ds((g % 8) * L, L)] = iid >> 1
        subs_u[pl.ds(g * L, L)] = (uid & 1) * EMB
        subs_v[pl.ds(g * L, L)] = (iid & 1) * EMB

    plsc.subcore_barrier()

    copies = []
    for j in range(NIDX):
        sl = pl.ds(j * IDXW, IDXW)
        copies.append(pltpu.async_copy(user_bias.at[idx_u.at[j]], bu.at[sl], sem))
        copies.append(pltpu.async_copy(item_bias.at[idx_v.at[j]], bi.at[sl], sem))
    for c in copies:
        c.wait()

    s_regs = []
    for k in range(KREG):
        sv0 = sv_v[0, pl.ds(k * L, L)]
        sv1 = sv_v[1, pl.ds(k * L, L)]
        s_regs.append(_sig(sv0 * 15.0) * _sig(sv1 * 15.0))
    mean16 = mean_v[pl.ds(0, L)]

    for h in range(2):
        hcopies = []
        for j in range(2):
            sl = pl.ds(j * IDXW, IDXW)
            r = 2 * h + j
            hcopies.append(pltpu.async_copy(user_emb.at[tiles_u.at[r]], u_rows.at[sl], sem))
            hcopies.append(pltpu.async_copy(item_emb.at[tiles_v.at[r]], i_rows.at[sl], sem))
        for c in hcopies:
            c.wait()

        def chunk_body(g, carry, h=h):
            cbase = h * (BPW // 2) + g * L
            dotv = jnp.zeros((L,), jnp.float32)
            for jj in range(L):
                e = cbase + jj
                l_vec = jnp.full((L,), g * L + jj, jnp.int32)
                e_vec = jnp.full((L,), e, jnp.int32)
                su = plsc.load_gather(subs_u, [e_vec])
                si = plsc.load_gather(subs_v, [e_vec])
                p = jnp.zeros((L,), jnp.float32)
                for k in range(KREG):
                    uk = plsc.load_gather(u_rows, [l_vec, su + cols[k]])
                    ik = plsc.load_gather(i_rows, [l_vec, si + cols[k]])
                    p = p + (uk * ik) * s_regs[k]
                dotv = jnp.where(lanes == jj, jnp.sum(p), dotv)
            bu16 = bu[pl.ds(cbase, L)]
            bi16 = bi[pl.ds(cbase, L)]
            out_v[pl.ds(cbase, L)] = _sig(dotv + bu16 + bi16 + mean16)
            return carry

        lax.fori_loop(0, CHUNKS // 2, chunk_body, 0)

    pltpu.sync_copy(out_v, out_hbm.at[pl.ds(wid * BPW, BPW)])


_MESH = plsc.VectorSubcoreMesh(
    core_axis_name="c", subcore_axis_name="s",
    num_cores=NUM_CORES, num_subcores=NUM_SUBCORES)

_MF = pl.kernel(
    _mf_body,
    out_type=jax.ShapeDtypeStruct((BATCH,), jnp.float32),
    mesh=_MESH,
    compiler_params=pltpu.CompilerParams(
        needs_layout_passes=False, use_tc_tiling_on_sc=False),
    scratch_types=[
        pltpu.VMEM((2 * BPW,), jnp.int32),
        pltpu.VMEM((NIDX, IDXW), jnp.int32),
        pltpu.VMEM((NIDX, IDXW), jnp.int32),
        pltpu.VMEM((NIDX, IDXW), jnp.int32),
        pltpu.VMEM((NIDX, IDXW), jnp.int32),
        pltpu.VMEM((BPW,), jnp.int32),
        pltpu.VMEM((BPW,), jnp.int32),
        pltpu.VMEM((BPW // 2, 2 * EMB), jnp.float32),
        pltpu.VMEM((BPW // 2, 2 * EMB), jnp.float32),
        pltpu.VMEM((BPW,), jnp.float32),
        pltpu.VMEM((BPW,), jnp.float32),
        pltpu.VMEM((L,), jnp.float32),
        pltpu.VMEM((2, EMB), jnp.float32),
        pltpu.VMEM((BPW,), jnp.float32),
        pltpu.SemaphoreType.DMA,
    ],
)


def kernel(fields, user_emb, user_bias, item_emb, item_bias, mean, sparse_var):
    fld = fields.reshape(-1)
    ue2 = user_emb.reshape(-1, 2 * EMB)
    ie2 = item_emb.reshape(-1, 2 * EMB)
    ub1 = user_bias.reshape(-1)
    ib1 = item_bias.reshape(-1)
    mean_vec = jnp.broadcast_to(mean, (L,))
    out = _MF(fld, ue2, ub1, ie2, ib1, mean_vec, sparse_var)
    dist = jnp.zeros((1,), dtype=jnp.float32)
    return (out, dist)


# speedup vs baseline: 1.0036x; 1.0036x over previous
import jax
import jax.numpy as jnp
from jax import lax
from jax.experimental import pallas as pl
from jax.experimental.pallas import tpu as pltpu
from jax.experimental.pallas import tpu_sc as plsc

NUM_CORES = 2
NUM_SUBCORES = 16
NW = NUM_CORES * NUM_SUBCORES
L = 16
BATCH = 16384
EMB = 64
BPW = BATCH // NW
NIDX = 4
IDXW = BPW // NIDX
CHUNKS = BPW // L
KREG = EMB // L


def _sig(x):
    return 1.0 / (1.0 + jnp.exp(-x))


def _mf_body(fld_hbm, user_emb, user_bias, item_emb, item_bias,
             mean_hbm, svar_hbm, out_hbm,
             fld_v, idx_u, idx_v, u_rows, i_rows, bu, bi, mean_v, sv_v,
             out_v, sem):
    wid = lax.axis_index("s") * NUM_CORES + lax.axis_index("c")

    pltpu.sync_copy(fld_hbm.at[pl.ds(wid * (2 * BPW), 2 * BPW)], fld_v)
    pltpu.sync_copy(mean_hbm, mean_v)
    pltpu.sync_copy(svar_hbm, sv_v)

    lanes = lax.iota(jnp.int32, L)
    zeros16 = jnp.zeros((L,), jnp.int32)
    mask15 = lanes == (L - 1)
    cols = [lanes + (k * L) for k in range(KREG)]

    for g in range(BPW // L):
        even = 2 * (lanes + g * L)
        idx_u[g // 8, pl.ds((g % 8) * L, L)] = plsc.load_gather(fld_v, [even])
        idx_v[g // 8, pl.ds((g % 8) * L, L)] = plsc.load_gather(fld_v, [even + 1])

    plsc.subcore_barrier()

    copies = []
    for j in range(NIDX):
        sl = pl.ds(j * IDXW, IDXW)
        copies.append(pltpu.async_copy(user_emb.at[idx_u.at[j]], u_rows.at[sl], sem))
        copies.append(pltpu.async_copy(item_emb.at[idx_v.at[j]], i_rows.at[sl], sem))
        copies.append(pltpu.async_copy(user_bias.at[idx_u.at[j]], bu.at[sl], sem))
        copies.append(pltpu.async_copy(item_bias.at[idx_v.at[j]], bi.at[sl], sem))
    for c in copies:
        c.wait()

    s_regs = []
    for k in range(KREG):
        sv0 = sv_v[0, pl.ds(k * L, L)]
        sv1 = sv_v[1, pl.ds(k * L, L)]
        s_regs.append(_sig(sv0 * 15.0) * _sig(sv1 * 15.0))
    mean16 = mean_v[pl.ds(0, L)]

    def chunk_body(g, carry):
        cbase = g * L
        dotv = jnp.zeros((L,), jnp.float32)
        for jj in range(L):
            b_vec = jnp.full((L,), cbase + jj, jnp.int32)
            p = jnp.zeros((L,), jnp.float32)
            for k in range(KREG):
                uk = plsc.load_gather(u_rows, [b_vec, cols[k]])
                ik = plsc.load_gather(i_rows, [b_vec, cols[k]])
                p = p + (uk * ik) * s_regs[k]
            dotv = jnp.where(lanes == jj, jnp.sum(p), dotv)
        bu16 = bu[pl.ds(cbase, L)]
        bi16 = bi[pl.ds(cbase, L)]
        out_v[pl.ds(cbase, L)] = _sig(dotv + bu16 + bi16 + mean16)
        return carry

    lax.fori_loop(0, CHUNKS, chunk_body, 0)

    pltpu.sync_copy(out_v, out_hbm.at[pl.ds(wid * BPW, BPW)])


_MESH = plsc.VectorSubcoreMesh(
    core_axis_name="c", subcore_axis_name="s",
    num_cores=NUM_CORES, num_subcores=NUM_SUBCORES)

_MF = pl.kernel(
    _mf_body,
    out_type=jax.ShapeDtypeStruct((BATCH,), jnp.float32),
    mesh=_MESH,
    compiler_params=pltpu.CompilerParams(
        needs_layout_passes=False, use_tc_tiling_on_sc=False),
    scratch_types=[
        pltpu.VMEM((2 * BPW,), jnp.int32),
        pltpu.VMEM((NIDX, IDXW), jnp.int32),
        pltpu.VMEM((NIDX, IDXW), jnp.int32),
        pltpu.VMEM((BPW, EMB), jnp.float32),
        pltpu.VMEM((BPW, EMB), jnp.float32),
        pltpu.VMEM((BPW,), jnp.float32),
        pltpu.VMEM((BPW,), jnp.float32),
        pltpu.VMEM((L,), jnp.float32),
        pltpu.VMEM((2, EMB), jnp.float32),
        pltpu.VMEM((BPW,), jnp.float32),
        pltpu.SemaphoreType.DMA,
    ],
)


def kernel(fields, user_emb, user_bias, item_emb, item_bias, mean, sparse_var):
    fld = fields.reshape(-1)
    ub1 = user_bias.reshape(-1)
    ib1 = item_bias.reshape(-1)
    mean_vec = jnp.broadcast_to(mean, (L,))
    out = _MF(fld, user_emb, ub1, item_emb, ib1, mean_vec, sparse_var)
    dist = jnp.zeros((1,), dtype=jnp.float32)
    return (out, dist)
